# baseline (speedup 1.0000x reference)
import functools

import jax
import jax.numpy as jnp
from jax import lax
from jax.experimental import pallas as pl
from jax.experimental.pallas import tpu as pltpu
from jax.experimental.pallas import tpu_sc as plsc

TYPES_NUM = 100000
DIM = 128

NW = 32
CHUNK = 200
NCHUNK = TYPES_NUM // CHUNK


@functools.partial(
    pl.kernel,
    mesh=plsc.VectorSubcoreMesh(core_axis_name="c", subcore_axis_name="s"),
    out_type=jax.ShapeDtypeStruct((TYPES_NUM, DIM), jnp.float32),
    scratch_types=[
        pltpu.VMEM((CHUNK, DIM), jnp.float32),
        pltpu.VMEM((CHUNK, DIM), jnp.float32),
        pltpu.SemaphoreType.DMA,
        pltpu.SemaphoreType.DMA,
    ],
)
def _sc_centers(bw_hbm, out_hbm, buf0, buf1, isem0, isem1):
    wid = lax.axis_index("s") * 2 + lax.axis_index("c")
    nmine = (NCHUNK - wid + NW - 1) // NW

    def src(k):
        r0 = (wid + k * NW) * CHUNK
        return bw_hbm.at[pl.ds(r0, CHUNK), pl.ds(0, DIM)]

    def dst(k):
        r0 = (wid + k * NW) * CHUNK
        return out_hbm.at[pl.ds(r0, CHUNK)]

    pltpu.async_copy(src(0), buf0, isem0)

    def body(p, carry):
        k0 = 2 * p
        k1 = k0 + 1

        pltpu.make_async_copy(src(k0), buf0, isem0).wait()

        @pl.when(k1 < nmine)
        def _():
            pltpu.async_copy(src(k1), buf1, isem1)

        pltpu.sync_copy(buf0, dst(k0))

        @pl.when(k1 < nmine)
        def _():
            pltpu.make_async_copy(src(k1), buf1, isem1).wait()

            @pl.when(k1 + 1 < nmine)
            def _():
                pltpu.async_copy(src(k1 + 1), buf0, isem0)

            pltpu.sync_copy(buf1, dst(k1))

        return carry

    lax.fori_loop(0, (nmine + 1) // 2, body, 0)


ROWS = 800


def _off_body(x_ref, o_ref):
    o_ref[...] = jnp.maximum(x_ref[...], 0.0) + 1e-6


def kernel(box_weight):
    n = box_weight.shape[0]
    centers = _sc_centers(box_weight)
    offsets = pl.pallas_call(
        _off_body,
        grid=(n // ROWS,),
        in_specs=[pl.BlockSpec((ROWS, DIM), lambda i: (i, 1))],
        out_specs=pl.BlockSpec((ROWS, DIM), lambda i: (i, 0)),
        out_shape=jax.ShapeDtypeStruct((n, DIM), jnp.float32),
    )(box_weight)
    return (centers, offsets)


# speedup vs baseline: 1.1039x; 1.1039x over previous
import functools

import jax
import jax.numpy as jnp
from jax import lax
from jax.experimental import pallas as pl
from jax.experimental.pallas import tpu as pltpu
from jax.experimental.pallas import tpu_sc as plsc

TYPES_NUM = 100000
DIM = 128

NW = 32
CHUNK = 200
NCHUNK = TYPES_NUM // CHUNK


@functools.partial(
    pl.kernel,
    mesh=plsc.VectorSubcoreMesh(core_axis_name="c", subcore_axis_name="s"),
    out_type=jax.ShapeDtypeStruct((TYPES_NUM, DIM), jnp.float32),
    scratch_types=[
        pltpu.VMEM((CHUNK, DIM), jnp.float32),
        pltpu.VMEM((CHUNK, DIM), jnp.float32),
        pltpu.SemaphoreType.DMA,
        pltpu.SemaphoreType.DMA,
    ],
)
def _sc_centers(bw_hbm, out_hbm, buf0, buf1, isem0, isem1):
    wid = lax.axis_index("s") * 2 + lax.axis_index("c")
    nmine = (NCHUNK - wid + NW - 1) // NW

    def src(k):
        r0 = (wid + k * NW) * CHUNK
        return bw_hbm.at[pl.ds(r0, CHUNK), pl.ds(0, DIM)]

    def dst(k):
        r0 = (wid + k * NW) * CHUNK
        return out_hbm.at[pl.ds(r0, CHUNK)]

    pltpu.async_copy(src(0), buf0, isem0)

    def body(p, carry):
        k0 = 2 * p
        k1 = k0 + 1

        pltpu.make_async_copy(src(k0), buf0, isem0).wait()

        @pl.when(k1 < nmine)
        def _():
            pltpu.async_copy(src(k1), buf1, isem1)

        pltpu.sync_copy(buf0, dst(k0))

        @pl.when(k1 < nmine)
        def _():
            pltpu.make_async_copy(src(k1), buf1, isem1).wait()

            @pl.when(k1 + 1 < nmine)
            def _():
                pltpu.async_copy(src(k1 + 1), buf0, isem0)

            pltpu.sync_copy(buf1, dst(k1))

        return carry

    lax.fori_loop(0, (nmine + 1) // 2, body, 0)


ROWS = 1000


def _off_body(x_ref, o_ref):
    o_ref[...] = jnp.maximum(x_ref[...], 0.0) + 1e-6


def kernel(box_weight):
    n = box_weight.shape[0]
    centers = _sc_centers(box_weight)
    offsets = pl.pallas_call(
        _off_body,
        grid=(n // ROWS,),
        in_specs=[pl.BlockSpec((ROWS, DIM), lambda i: (i, 1))],
        out_specs=pl.BlockSpec((ROWS, DIM), lambda i: (i, 0)),
        out_shape=jax.ShapeDtypeStruct((n, DIM), jnp.float32),
    )(box_weight)
    return (centers, offsets)


# speedup vs baseline: 1.1083x; 1.0040x over previous
import functools

import jax
import jax.numpy as jnp
from jax import lax
from jax.experimental import pallas as pl
from jax.experimental.pallas import tpu as pltpu
from jax.experimental.pallas import tpu_sc as plsc

TYPES_NUM = 100000
DIM = 128

NW = 32
CHUNK = 200
NCHUNK = TYPES_NUM // CHUNK
C0_CHUNKS = 300


@functools.partial(
    pl.kernel,
    mesh=plsc.VectorSubcoreMesh(core_axis_name="c", subcore_axis_name="s"),
    out_type=jax.ShapeDtypeStruct((TYPES_NUM, DIM), jnp.float32),
    scratch_types=[
        pltpu.VMEM((CHUNK, DIM), jnp.float32),
        pltpu.VMEM((CHUNK, DIM), jnp.float32),
        pltpu.SemaphoreType.DMA,
        pltpu.SemaphoreType.DMA,
    ],
)
def _sc_centers(bw_hbm, out_hbm, buf0, buf1, isem0, isem1):
    c = lax.axis_index("c")
    s = lax.axis_index("s")
    base = jnp.where(c == 0, 0, C0_CHUNKS)
    span = jnp.where(c == 0, C0_CHUNKS, NCHUNK - C0_CHUNKS)
    nmine = (span - s + 15) // 16

    def src(k):
        r0 = (base + s + k * 16) * CHUNK
        return bw_hbm.at[pl.ds(r0, CHUNK), pl.ds(0, DIM)]

    def dst(k):
        r0 = (base + s + k * 16) * CHUNK
        return out_hbm.at[pl.ds(r0, CHUNK)]

    pltpu.async_copy(src(0), buf0, isem0)

    def body(p, carry):
        k0 = 2 * p
        k1 = k0 + 1

        pltpu.make_async_copy(src(k0), buf0, isem0).wait()

        @pl.when(k1 < nmine)
        def _():
            pltpu.async_copy(src(k1), buf1, isem1)

        pltpu.sync_copy(buf0, dst(k0))

        @pl.when(k1 < nmine)
        def _():
            pltpu.make_async_copy(src(k1), buf1, isem1).wait()

            @pl.when(k1 + 1 < nmine)
            def _():
                pltpu.async_copy(src(k1 + 1), buf0, isem0)

            pltpu.sync_copy(buf1, dst(k1))

        return carry

    lax.fori_loop(0, (nmine + 1) // 2, body, 0)


ROWS = 1000


def _off_body(x_ref, o_ref):
    o_ref[...] = jnp.maximum(x_ref[...], 0.0) + 1e-6


def kernel(box_weight):
    n = box_weight.shape[0]
    centers = _sc_centers(box_weight)
    offsets = pl.pallas_call(
        _off_body,
        grid=(n // ROWS,),
        in_specs=[pl.BlockSpec((ROWS, DIM), lambda i: (i, 1))],
        out_specs=pl.BlockSpec((ROWS, DIM), lambda i: (i, 0)),
        out_shape=jax.ShapeDtypeStruct((n, DIM), jnp.float32),
    )(box_weight)
    return (centers, offsets)
